# baseline (device time: 10535 ns/iter reference)
import contextlib
import os

import jax
import jax.numpy as jnp
from jax import lax
from jax.experimental import pallas as pl
from jax.experimental.pallas import tpu as pltpu

N_DEV = 8

if os.environ.get("KPROF") == "1":
    _scope = jax.named_scope
else:
    def _scope(_name):
        return contextlib.nullcontext()


def kernel(x, w_mat):
    m_total, k_shard = x.shape
    k_total, n = w_mat.shape
    m_blk = m_total // N_DEV
    k_blk = k_total // N_DEV
    k_half = k_total // 2

    def body(x_ref, w_ref, out_ref, xv_ref, wf_ref, xb_ref, xg_ref, wb_ref,
             send_sems, recv_sems, xcp_sem, wcp_sems):
        my = lax.axis_index("i")

        with _scope("stage_start"):
            xcp = pltpu.make_async_copy(x_ref, xv_ref, xcp_sem)
            xcp.start()
            wcps = []
            for e in range(N_DEV):
                src = (my - e) % N_DEV
                wcp = pltpu.make_async_copy(
                    w_ref.at[pl.ds(src * k_blk, k_blk), :],
                    wf_ref.at[pl.ds(e * k_blk, k_blk), :],
                    wcp_sems.at[e],
                )
                wcp.start()
                wcps.append(wcp)

            barrier = pltpu.get_barrier_semaphore()
            for d in range(1, N_DEV):
                pl.semaphore_signal(
                    barrier, inc=1,
                    device_id=((my + d) % N_DEV,),
                    device_id_type=pl.DeviceIdType.MESH,
                )

        with _scope("xconv"):
            xcp.wait()
            xb_ref[...] = xv_ref[...].astype(jnp.bfloat16)

        with _scope("barrier_wait"):
            pl.semaphore_wait(barrier, N_DEV - 1)

        with _scope("rdma_start"):
            sends = []
            recvs = [None]
            for d in range(1, N_DEV):
                t = (my + d) % N_DEV
                rdma = pltpu.make_async_remote_copy(
                    src_ref=xb_ref.at[pl.ds(t * m_blk, m_blk), :],
                    dst_ref=xg_ref.at[:, pl.ds(d * k_blk, k_blk)],
                    send_sem=send_sems.at[d],
                    recv_sem=recv_sems.at[d],
                    device_id=(t,),
                    device_id_type=pl.DeviceIdType.MESH,
                )
                rdma.start()
                sends.append(rdma)
                recvs.append(rdma)

            xg_ref[:, pl.ds(0, k_blk)] = xb_ref[pl.ds(my * m_blk, m_blk), :]

        with _scope("half0"):
            for e in range(N_DEV // 2):
                wcps[e].wait()
                wb_ref[pl.ds(e * k_blk, k_blk), :] = (
                    wf_ref[pl.ds(e * k_blk, k_blk), :].astype(jnp.bfloat16)
                )
            for d in range(1, N_DEV // 2):
                recvs[d].wait_recv()
            out_ref[...] = jnp.dot(
                xg_ref[:, :k_half], wb_ref[:k_half, :],
                preferred_element_type=jnp.float32,
            )

        with _scope("half1"):
            for e in range(N_DEV // 2, N_DEV):
                wcps[e].wait()
                wb_ref[pl.ds(e * k_blk, k_blk), :] = (
                    wf_ref[pl.ds(e * k_blk, k_blk), :].astype(jnp.bfloat16)
                )
            for d in range(N_DEV // 2, N_DEV):
                recvs[d].wait_recv()
            out_ref[...] += jnp.dot(
                xg_ref[:, k_half:], wb_ref[k_half:, :],
                preferred_element_type=jnp.float32,
            )

        with _scope("drain"):
            for rdma in sends:
                rdma.wait_send()

    x = pltpu.with_memory_space_constraint(x, pltpu.MemorySpace.HBM)
    w_mat = pltpu.with_memory_space_constraint(w_mat, pltpu.MemorySpace.HBM)

    return pl.pallas_call(
        body,
        out_shape=jax.ShapeDtypeStruct((m_blk, n), jnp.float32),
        in_specs=[
            pl.BlockSpec(memory_space=pltpu.MemorySpace.HBM),
            pl.BlockSpec(memory_space=pltpu.MemorySpace.HBM),
        ],
        out_specs=pl.BlockSpec(memory_space=pltpu.VMEM),
        scratch_shapes=[
            pltpu.VMEM((m_total, k_shard), jnp.float32),
            pltpu.VMEM((k_total, n), jnp.float32),
            pltpu.VMEM((m_total, k_shard), jnp.bfloat16),
            pltpu.VMEM((m_blk, k_total), jnp.bfloat16),
            pltpu.VMEM((k_total, n), jnp.bfloat16),
            pltpu.SemaphoreType.DMA((N_DEV,)),
            pltpu.SemaphoreType.DMA((N_DEV,)),
            pltpu.SemaphoreType.DMA,
            pltpu.SemaphoreType.DMA((N_DEV,)),
        ],
        compiler_params=pltpu.CompilerParams(collective_id=0),
    )(x, w_mat)


# device time: 9647 ns/iter; 1.0920x vs baseline; 1.0920x over previous
import contextlib
import os

import jax
import jax.numpy as jnp
from jax import lax
from jax.experimental import pallas as pl
from jax.experimental.pallas import tpu as pltpu

N_DEV = 8

if os.environ.get("KPROF") == "1":
    _scope = jax.named_scope
else:
    def _scope(_name):
        return contextlib.nullcontext()

_ABL = os.environ.get("KABL", "")
_DO_COMM = _ABL in ("", "auto") or _ABL.startswith("comm")
_DO_BARRIER = _ABL in ("", "barrier") or _ABL.startswith("comm")
_N_COMM = 8 if _ABL in ("", "auto") else (
    int(_ABL[4:]) + 1 if _ABL.startswith("comm") else 1
)


def kernel(x, w_mat):
    m_total, k_shard = x.shape
    k_total, n = w_mat.shape
    m_blk = m_total // N_DEV
    k_blk = k_total // N_DEV
    k_half = k_total // 2

    def body(x_ref, w_ref, out_ref, xv_ref, wf_ref, xb_ref, xg_ref, wb_ref,
             send_sems, recv_sems, xcp_sem, wcp_sems):
        my = lax.axis_index("i")

        with _scope("stage_start"):
            xcp = pltpu.make_async_copy(x_ref, xv_ref, xcp_sem)
            xcp.start()

            if _DO_BARRIER:
                barrier = pltpu.get_barrier_semaphore()
                for d in range(1, N_DEV):
                    pl.semaphore_signal(
                        barrier, inc=1,
                        device_id=((my + d) % N_DEV,),
                        device_id_type=pl.DeviceIdType.MESH,
                    )

        with _scope("xconv"):
            xcp.wait()
            xb_ref[...] = xv_ref[...].astype(jnp.bfloat16)

        with _scope("barrier_wait"):
            if _DO_BARRIER:
                pl.semaphore_wait(barrier, N_DEV - 1)

        with _scope("rdma_start"):
            sends = []
            recvs = [None]
            for d in range(1, _N_COMM) if _DO_COMM else []:
                t = (my + d) % N_DEV
                rdma = pltpu.make_async_remote_copy(
                    src_ref=xb_ref.at[pl.ds(t * m_blk, m_blk), :],
                    dst_ref=xg_ref.at[d],
                    send_sem=send_sems.at[d],
                    recv_sem=recv_sems.at[d],
                    device_id=(t,),
                    device_id_type=pl.DeviceIdType.MESH,
                )
                rdma.start()
                sends.append(rdma)
                recvs.append(rdma)

            xg_ref[0] = xb_ref[pl.ds(my * m_blk, m_blk), :]

        with _scope("w_stage"):
            wcps = []
            for e in range(N_DEV):
                src = (my - e) % N_DEV
                wcp = pltpu.make_async_copy(
                    w_ref.at[pl.ds(src * k_blk, k_blk), :],
                    wf_ref.at[pl.ds(e * k_blk, k_blk), :],
                    wcp_sems.at[e],
                )
                wcp.start()
                wcps.append(wcp)

        for d in range(N_DEV):
            with _scope(f"slot{d}"):
                wcps[d].wait()
                wb_ref[pl.ds(d * k_blk, k_blk), :] = (
                    wf_ref[pl.ds(d * k_blk, k_blk), :].astype(jnp.bfloat16)
                )
                if 0 < d < _N_COMM and _DO_COMM:
                    recvs[d].wait_recv()
                part = jnp.dot(
                    xg_ref[d], wb_ref[pl.ds(d * k_blk, k_blk), :],
                    preferred_element_type=jnp.float32,
                )
                if d == 0:
                    out_ref[...] = part
                else:
                    out_ref[...] += part

        with _scope("drain"):
            for rdma in sends:
                rdma.wait_send()

    x = pltpu.with_memory_space_constraint(x, pltpu.MemorySpace.HBM)
    w_mat = pltpu.with_memory_space_constraint(w_mat, pltpu.MemorySpace.HBM)

    return pl.pallas_call(
        body,
        out_shape=jax.ShapeDtypeStruct((m_blk, n), jnp.float32),
        in_specs=[
            pl.BlockSpec(memory_space=pltpu.MemorySpace.HBM),
            pl.BlockSpec(memory_space=pltpu.MemorySpace.HBM),
        ],
        out_specs=pl.BlockSpec(memory_space=pltpu.VMEM),
        scratch_shapes=[
            pltpu.VMEM((m_total, k_shard), jnp.float32),
            pltpu.VMEM((k_total, n), jnp.float32),
            pltpu.VMEM((m_total, k_shard), jnp.bfloat16),
            pltpu.VMEM((N_DEV, m_blk, k_blk), jnp.bfloat16),
            pltpu.VMEM((k_total, n), jnp.bfloat16),
            pltpu.SemaphoreType.DMA((N_DEV,)),
            pltpu.SemaphoreType.DMA((N_DEV,)),
            pltpu.SemaphoreType.DMA,
            pltpu.SemaphoreType.DMA((N_DEV,)),
        ],
        compiler_params=pltpu.CompilerParams(
            collective_id=0 if _DO_BARRIER else None
        ),
    )(x, w_mat)
